# baseline (speedup 1.0000x reference)
import functools

import jax
import jax.numpy as jnp
from jax import lax
from jax.experimental import pallas as pl
from jax.experimental.pallas import tpu as pltpu
from jax.experimental.pallas import tpu_sc as plsc

_N = 1024
_D = 128
_P = 4096
_R = 3
_K = 16
_RK = _R * _K
_CURRENT_TIME = 200.0

_NW = 32
_NODES_W = _N // _NW
_PAIRS_W = _P // _NW
_NCHUNK = 4
_NSTEPS = _NODES_W // _NCHUNK


def _prep_stage(ev_ref, E_ref, Wp_ref, Wb_ref, bb_ref, theta_ref,
                proj_ref, ptab_ref, bbar_ref, alpha_ref):
    E = E_ref[...]
    proj_ref[...] = jnp.dot(E, Wp_ref[...], preferred_element_type=jnp.float32)
    for r in range(_R):
        ptab_ref[r * _N:(r + 1) * _N, :] = jnp.dot(
            E, Wb_ref[r], preferred_element_type=jnp.float32)
    bbar_ref[...] = jnp.mean(bb_ref[...], axis=0, keepdims=True)
    theta = theta_ref[0, 0]
    alpha_ref[...] = jnp.sum(
        jnp.exp(-theta * (_CURRENT_TIME - ev_ref[...]))).reshape(1, 1)


def _make_neigh_kernel():
    mesh = plsc.VectorSubcoreMesh(core_axis_name="c", subcore_axis_name="s")
    rows = _NCHUNK * _RK

    @functools.partial(
        pl.kernel, mesh=mesh,
        compiler_params=pltpu.CompilerParams(needs_layout_passes=False),
        out_type=jax.ShapeDtypeStruct((_N,), jnp.float32),
        scratch_types=[
            pltpu.VMEM((_NODES_W * _RK,), jnp.int32),
            pltpu.VMEM((rows, _D), jnp.float32),
            pltpu.VMEM((rows, _D), jnp.float32),
            pltpu.VMEM((_D,), jnp.float32),
            pltpu.VMEM((_NODES_W,), jnp.float32),
            pltpu.VMEM((16, 16), jnp.float32),
            pltpu.SemaphoreType.DMA,
            pltpu.SemaphoreType.DMA,
        ],
    )
    def neigh(ptab_hbm, idx_hbm, bbar_hbm, s_hbm,
              idx_v, rows0, rows1, bbar_v, s_v, totm, sem0, sem1):
        wid = lax.axis_index("s") * 2 + lax.axis_index("c")
        base = wid * (_NODES_W * _RK)
        pltpu.sync_copy(idx_hbm.at[pl.ds(base, _NODES_W * _RK)], idx_v)
        pltpu.sync_copy(bbar_hbm, bbar_v)

        bufs = (rows0, rows1)
        sems = (sem0, sem1)

        def fire(t):
            return pltpu.async_copy(
                ptab_hbm.at[idx_v.at[pl.ds(t * rows, rows)]],
                bufs[t % 2], sems[t % 2])

        lane = lax.broadcasted_iota(jnp.int32, (16,), 0)
        cps = {0: fire(0), 1: fire(1)}
        for t in range(_NSTEPS):
            cps[t].wait()
            buf = bufs[t % 2]
            for u in range(_NCHUNK):
                def row_body(j, accs, _u=u, _buf=buf):
                    out = list(accs)
                    for jj in range(4):
                        row = _u * _RK + j * 4 + jj
                        for c in range(8):
                            out[c] = out[c] + _buf[row, pl.ds(c * 16, 16)]
                    return tuple(out)
                accs = tuple(jnp.zeros((16,), jnp.float32) for _ in range(8))
                accs = lax.fori_loop(0, _RK // 4, row_body, accs)
                tot = jnp.zeros((16,), jnp.float32)
                for c in range(8):
                    x = accs[c] * (1.0 / _RK) + bbar_v[pl.ds(c * 16, 16)]
                    tot = tot + 1.0 / (1.0 + jnp.exp(-x))
                node = t * _NCHUNK + u
                totm[node % 16, pl.ds(0, 16)] = tot * (1.0 / _D)
                if node % 16 == 15:
                    acc = jnp.zeros((16,), jnp.float32)
                    for c in range(16):
                        acc = acc + plsc.load_gather(
                            totm, [lane, jnp.full((16,), c, jnp.int32)])
                    s_v[pl.ds((node // 16) * 16, 16)] = acc
            if t + 2 < _NSTEPS:
                cps[t + 2] = fire(t + 2)
        pltpu.sync_copy(s_v, s_hbm.at[pl.ds(wid * _NODES_W, _NODES_W)])

    return neigh


def _make_pair_kernel():
    mesh = plsc.VectorSubcoreMesh(core_axis_name="c", subcore_axis_name="s")
    half = _PAIRS_W

    @functools.partial(
        pl.kernel, mesh=mesh,
        compiler_params=pltpu.CompilerParams(needs_layout_passes=False),
        out_type=[jax.ShapeDtypeStruct((_P,), jnp.float32),
                  jax.ShapeDtypeStruct((_P,), jnp.float32)],
        scratch_types=[
            pltpu.VMEM((2 * _PAIRS_W,), jnp.int32),
            pltpu.VMEM((_PAIRS_W,), jnp.int32),
            pltpu.VMEM((_PAIRS_W,), jnp.int32),
            pltpu.VMEM((_N,), jnp.float32),
            pltpu.VMEM((2 * _PAIRS_W, _D), jnp.float32),
            pltpu.VMEM((_PAIRS_W,), jnp.float32),
            pltpu.VMEM((_PAIRS_W,), jnp.float32),
            pltpu.SemaphoreType.DMA,
            pltpu.SemaphoreType.DMA,
        ],
    )
    def pair(proj_hbm, mn_hbm, m_hbm, n_hbm, s_hbm, d2_hbm, lamp_hbm,
             mn_v, m_v, n_v, s_v, rows_v, d2_v, lamp_v, sem0, sem1):
        wid = lax.axis_index("s") * 2 + lax.axis_index("c")
        pltpu.sync_copy(mn_hbm.at[pl.ds(wid * 2 * _PAIRS_W, 2 * _PAIRS_W)],
                        mn_v)
        cp0 = pltpu.async_copy(proj_hbm.at[mn_v.at[pl.ds(0, half)]],
                               rows_v.at[pl.ds(0, half)], sem0)
        cp1 = pltpu.async_copy(proj_hbm.at[mn_v.at[pl.ds(half, half)]],
                               rows_v.at[pl.ds(half, half)], sem1)
        pltpu.sync_copy(m_hbm.at[pl.ds(wid * _PAIRS_W, _PAIRS_W)], m_v)
        pltpu.sync_copy(n_hbm.at[pl.ds(wid * _PAIRS_W, _PAIRS_W)], n_v)
        pltpu.sync_copy(s_hbm, s_v)
        for g in range(_PAIRS_W // 16):
            mi = m_v[pl.ds(g * 16, 16)]
            ni = n_v[pl.ds(g * 16, 16)]
            sm = plsc.load_gather(s_v, [mi])
            sn = plsc.load_gather(s_v, [ni])
            lamp_v[pl.ds(g * 16, 16)] = 0.5 * (sm + sn)
        cp0.wait()
        cp1.wait()

        lane = lax.broadcasted_iota(jnp.int32, (16,), 0)
        for g in range(_PAIRS_W // 16):
            rowm = 2 * (g * 16 + lane)
            rown = rowm + 1

            def col_body(d, acc, _rowm=rowm, _rown=rown):
                for dd in range(2):
                    colv = jnp.full((16,), d * 2 + dd, jnp.int32)
                    a = plsc.load_gather(rows_v, [_rowm, colv])
                    b = plsc.load_gather(rows_v, [_rown, colv])
                    dm = a - b
                    acc = acc + dm * dm
                return acc

            d2g = lax.fori_loop(0, _D // 2, col_body,
                                jnp.zeros((16,), jnp.float32))
            d2_v[pl.ds(g * 16, 16)] = d2g
        pltpu.sync_copy(d2_v, d2_hbm.at[pl.ds(wid * _PAIRS_W, _PAIRS_W)])
        pltpu.sync_copy(lamp_v, lamp_hbm.at[pl.ds(wid * _PAIRS_W, _PAIRS_W)])

    return pair


_neigh_sc = _make_neigh_kernel()
_pair_sc = _make_pair_kernel()


def _tail_stage(d2_ref, lamp_ref, alpha_ref, q1_ref, q2_ref, out_ref):
    lam = -jnp.sqrt(d2_ref[...] + 1e-12) + alpha_ref[0, 0] + lamp_ref[...]
    y = q1_ref[0, 0] * jnp.exp(lam) + q2_ref[0, 0] * lam
    out_ref[...] = jax.nn.sigmoid(y)


def kernel(node_pairs, adj_matrix, event_history, neighbor_data, node_embeds,
           W_proj, W_beta, b_beta, decay_theta, q1, q2):
    del adj_matrix
    f32 = jnp.float32
    i32 = jnp.int32
    T = event_history.shape[0]
    tpad = (-T) % _D
    ev = jnp.concatenate(
        [event_history.astype(f32), jnp.full((tpad,), -1e30, f32)])
    ev = ev.reshape(-1, _D)
    theta = jnp.reshape(decay_theta.astype(f32), (1, 1))
    q1r = jnp.reshape(jnp.asarray(q1, f32), (1, 1))
    q2r = jnp.reshape(jnp.asarray(q2, f32), (1, 1))

    offs = (jnp.arange(_R, dtype=i32) * _N)[None, :, None]
    idxp = (neighbor_data.astype(i32) + offs).reshape(-1)
    pairs = node_pairs.astype(i32)
    mnflat = pairs.reshape(-1)
    m = pairs[:, 0]
    n = pairs[:, 1]

    proj, ptab, bbar, alpha = pl.pallas_call(
        _prep_stage,
        out_shape=[
            jax.ShapeDtypeStruct((_N, _D), f32),
            jax.ShapeDtypeStruct((_R * _N, _D), f32),
            jax.ShapeDtypeStruct((1, _D), f32),
            jax.ShapeDtypeStruct((1, 1), f32),
        ],
    )(ev, node_embeds.astype(f32), W_proj.astype(f32), W_beta.astype(f32),
      b_beta.astype(f32), theta)

    s = _neigh_sc(ptab, idxp, bbar.reshape(_D))
    d2, lamp = _pair_sc(proj, mnflat, m, n, s)

    out = pl.pallas_call(
        _tail_stage,
        out_shape=jax.ShapeDtypeStruct((_NW, _D), f32),
    )(d2.reshape(_NW, _D), lamp.reshape(_NW, _D), alpha, q1r, q2r)
    return out.reshape(_P)


# speedup vs baseline: 1.0122x; 1.0122x over previous
import functools

import jax
import jax.numpy as jnp
from jax import lax
from jax.experimental import pallas as pl
from jax.experimental.pallas import tpu as pltpu
from jax.experimental.pallas import tpu_sc as plsc

_N = 1024
_D = 128
_P = 4096
_R = 3
_K = 16
_RK = _R * _K
_CURRENT_TIME = 200.0

_NW = 32
_NODES_W = _N // _NW
_PAIRS_W = _P // _NW
_NCHUNK = 4
_NSTEPS = _NODES_W // _NCHUNK
_PAIR_BLK = 512


def _prep_stage(ev_ref, E_ref, Wp_ref, Wb_ref, bb_ref, theta_ref,
                proj_ref, ptab_ref, bbar_ref, alpha_ref):
    E = E_ref[...]
    proj_ref[...] = jnp.dot(E, Wp_ref[...], preferred_element_type=jnp.float32)
    for r in range(_R):
        ptab_ref[r * _N:(r + 1) * _N, :] = jnp.dot(
            E, Wb_ref[r], preferred_element_type=jnp.float32)
    bbar_ref[...] = jnp.mean(bb_ref[...], axis=0, keepdims=True)
    theta = theta_ref[0, 0]
    alpha_ref[...] = jnp.sum(
        jnp.exp(-theta * (_CURRENT_TIME - ev_ref[...]))).reshape(1, 1)


def _make_sc_kernel():
    mesh = plsc.VectorSubcoreMesh(core_axis_name="c", subcore_axis_name="s")
    rows = _NCHUNK * _RK

    @functools.partial(
        pl.kernel, mesh=mesh,
        compiler_params=pltpu.CompilerParams(needs_layout_passes=False),
        out_type=[jax.ShapeDtypeStruct((_N,), jnp.float32),
                  jax.ShapeDtypeStruct((_P,), jnp.float32)],
        scratch_types=[
            pltpu.VMEM((_NODES_W * _RK,), jnp.int32),
            pltpu.VMEM((rows, _D), jnp.float32),
            pltpu.VMEM((rows, _D), jnp.float32),
            pltpu.VMEM((_D,), jnp.float32),
            pltpu.VMEM((_NODES_W,), jnp.float32),
            pltpu.VMEM((16, 16), jnp.float32),
            pltpu.VMEM((2 * _PAIRS_W,), jnp.int32),
            pltpu.VMEM((2 * _PAIRS_W, _D), jnp.float32),
            pltpu.VMEM((_PAIRS_W,), jnp.float32),
            pltpu.SemaphoreType.DMA,
            pltpu.SemaphoreType.DMA,
            pltpu.SemaphoreType.DMA,
            pltpu.SemaphoreType.DMA,
        ],
    )
    def sc_body(ptab_hbm, idx_hbm, bbar_hbm, proj_hbm, mn_hbm,
                s_hbm, d2_hbm,
                idx_v, rows0, rows1, bbar_v, s_v, totm, mn_v, prow_v, d2_v,
                sem0, sem1, semp0, semp1):
        wid = lax.axis_index("s") * 2 + lax.axis_index("c")
        base = wid * (_NODES_W * _RK)
        pltpu.sync_copy(idx_hbm.at[pl.ds(base, _NODES_W * _RK)], idx_v)
        pltpu.sync_copy(mn_hbm.at[pl.ds(wid * 2 * _PAIRS_W, 2 * _PAIRS_W)],
                        mn_v)
        pltpu.sync_copy(bbar_hbm, bbar_v)
        cpp0 = pltpu.async_copy(proj_hbm.at[mn_v.at[pl.ds(0, _PAIRS_W)]],
                                prow_v.at[pl.ds(0, _PAIRS_W)], semp0)
        cpp1 = pltpu.async_copy(
            proj_hbm.at[mn_v.at[pl.ds(_PAIRS_W, _PAIRS_W)]],
            prow_v.at[pl.ds(_PAIRS_W, _PAIRS_W)], semp1)

        bufs = (rows0, rows1)
        sems = (sem0, sem1)

        def fire(t):
            return pltpu.async_copy(
                ptab_hbm.at[idx_v.at[pl.ds(t * rows, rows)]],
                bufs[t % 2], sems[t % 2])

        lane = lax.broadcasted_iota(jnp.int32, (16,), 0)

        def transpose_sum(dst, off):
            acc = jnp.zeros((16,), jnp.float32)
            for c in range(16):
                acc = acc + plsc.load_gather(
                    totm, [lane, jnp.full((16,), c, jnp.int32)])
            dst[pl.ds(off, 16)] = acc

        cps = {0: fire(0), 1: fire(1)}
        for t in range(_NSTEPS):
            cps[t].wait()
            buf = bufs[t % 2]
            for u in range(_NCHUNK):
                def row_body(j, accs, _u=u, _buf=buf):
                    out = list(accs)
                    for jj in range(4):
                        row = _u * _RK + j * 4 + jj
                        for c in range(8):
                            out[c] = out[c] + _buf[row, pl.ds(c * 16, 16)]
                    return tuple(out)
                accs = tuple(jnp.zeros((16,), jnp.float32) for _ in range(8))
                accs = lax.fori_loop(0, _RK // 4, row_body, accs)
                tot = jnp.zeros((16,), jnp.float32)
                for c in range(8):
                    x = accs[c] * (1.0 / _RK) + bbar_v[pl.ds(c * 16, 16)]
                    tot = tot + 1.0 / (1.0 + jnp.exp(-x))
                node = t * _NCHUNK + u
                totm[node % 16, pl.ds(0, 16)] = tot * (1.0 / _D)
                if node % 16 == 15:
                    transpose_sum(s_v, (node // 16) * 16)
            if t + 2 < _NSTEPS:
                cps[t + 2] = fire(t + 2)
        pltpu.sync_copy(s_v, s_hbm.at[pl.ds(wid * _NODES_W, _NODES_W)])

        cpp0.wait()
        cpp1.wait()
        for g in range(_PAIRS_W // 16):
            def pair_body(q, _, _g=g):
                p = _g * 16 + q
                acc = jnp.zeros((16,), jnp.float32)
                for c in range(8):
                    a = prow_v[2 * p, pl.ds(c * 16, 16)]
                    b = prow_v[2 * p + 1, pl.ds(c * 16, 16)]
                    dm = a - b
                    acc = acc + dm * dm
                totm[q, pl.ds(0, 16)] = acc
                return 0
            lax.fori_loop(0, 16, pair_body, 0)
            transpose_sum(d2_v, g * 16)
        pltpu.sync_copy(d2_v, d2_hbm.at[pl.ds(wid * _PAIRS_W, _PAIRS_W)])

    return sc_body


_sc_stage = _make_sc_kernel()


def _tail_stage(m_ref, n_ref, s_ref, d2_ref, alpha_ref, q1_ref, q2_ref,
                out_ref):
    iota = lax.broadcasted_iota(jnp.int32, (_PAIR_BLK, _N), 1)
    ohm = (m_ref[...] == iota).astype(jnp.float32)
    ohn = (n_ref[...] == iota).astype(jnp.float32)
    lamn = 0.5 * jnp.dot(ohm + ohn, s_ref[...],
                         preferred_element_type=jnp.float32)
    lam = -jnp.sqrt(d2_ref[...] + 1e-12) + alpha_ref[0, 0] + lamn
    y = q1_ref[0, 0] * jnp.exp(lam) + q2_ref[0, 0] * lam
    out_ref[...] = jax.nn.sigmoid(y)


def kernel(node_pairs, adj_matrix, event_history, neighbor_data, node_embeds,
           W_proj, W_beta, b_beta, decay_theta, q1, q2):
    del adj_matrix
    f32 = jnp.float32
    i32 = jnp.int32
    T = event_history.shape[0]
    tpad = (-T) % _D
    ev = jnp.concatenate(
        [event_history.astype(f32), jnp.full((tpad,), -1e30, f32)])
    ev = ev.reshape(-1, _D)
    theta = jnp.reshape(decay_theta.astype(f32), (1, 1))
    q1r = jnp.reshape(jnp.asarray(q1, f32), (1, 1))
    q2r = jnp.reshape(jnp.asarray(q2, f32), (1, 1))

    offs = (jnp.arange(_R, dtype=i32) * _N)[None, :, None]
    idxp = (neighbor_data.astype(i32) + offs).reshape(-1)
    pairs = node_pairs.astype(i32)
    mnflat = pairs.reshape(-1)
    m = pairs[:, 0:1]
    n = pairs[:, 1:2]

    proj, ptab, bbar, alpha = pl.pallas_call(
        _prep_stage,
        out_shape=[
            jax.ShapeDtypeStruct((_N, _D), f32),
            jax.ShapeDtypeStruct((_R * _N, _D), f32),
            jax.ShapeDtypeStruct((1, _D), f32),
            jax.ShapeDtypeStruct((1, 1), f32),
        ],
    )(ev, node_embeds.astype(f32), W_proj.astype(f32), W_beta.astype(f32),
      b_beta.astype(f32), theta)

    s, d2 = _sc_stage(ptab, idxp, bbar.reshape(_D), proj, mnflat)

    nblk = _P // _PAIR_BLK
    out = pl.pallas_call(
        _tail_stage,
        grid=(nblk,),
        in_specs=[
            pl.BlockSpec((_PAIR_BLK, 1), lambda i: (i, 0)),
            pl.BlockSpec((_PAIR_BLK, 1), lambda i: (i, 0)),
            pl.BlockSpec((_N, 1), lambda i: (0, 0)),
            pl.BlockSpec((_PAIR_BLK, 1), lambda i: (i, 0)),
            pl.BlockSpec((1, 1), lambda i: (0, 0)),
            pl.BlockSpec((1, 1), lambda i: (0, 0)),
            pl.BlockSpec((1, 1), lambda i: (0, 0)),
        ],
        out_specs=pl.BlockSpec((_PAIR_BLK, 1), lambda i: (i, 0)),
        out_shape=jax.ShapeDtypeStruct((_P, 1), f32),
    )(m, n, s.reshape(_N, 1), d2.reshape(_P, 1), alpha, q1r, q2r)
    return out.reshape(_P)


# speedup vs baseline: 1.2141x; 1.1994x over previous
import functools

import jax
import jax.numpy as jnp
from jax import lax
from jax.experimental import pallas as pl
from jax.experimental.pallas import tpu as pltpu
from jax.experimental.pallas import tpu_sc as plsc

_N = 1024
_D = 128
_P = 4096
_R = 3
_K = 16
_RK = _R * _K
_CURRENT_TIME = 200.0

_NW = 32
_NODES_W = _N // _NW
_PAIRS_W = _P // _NW
_NSTEP = 4
_NPS = _NODES_W // _NSTEP
_ROWS = _NPS * _RK
_PAIR_BLK = 512


def _make_sc_kernel():
    mesh = plsc.VectorSubcoreMesh(core_axis_name="c", subcore_axis_name="s")

    @functools.partial(
        pl.kernel, mesh=mesh,
        compiler_params=pltpu.CompilerParams(
            needs_layout_passes=False, use_tc_tiling_on_sc=False),
        out_type=[jax.ShapeDtypeStruct((_R * _N, _D), jnp.float32),
                  jax.ShapeDtypeStruct((_P, _D), jnp.float32)],
        scratch_types=[
            pltpu.VMEM((_NODES_W * _RK,), jnp.int32),
            pltpu.VMEM((_ROWS, _D // 2), jnp.int32),
            pltpu.VMEM((_ROWS, _D // 2), jnp.int32),
            pltpu.VMEM((2 * _PAIRS_W,), jnp.int32),
            pltpu.VMEM((2 * _PAIRS_W, _D), jnp.float32),
            pltpu.VMEM((_PAIRS_W, _D), jnp.float32),
            pltpu.VMEM((_R * _NODES_W, _D), jnp.float32),
            pltpu.SemaphoreType.DMA,
            pltpu.SemaphoreType.DMA,
            pltpu.SemaphoreType.DMA,
            pltpu.SemaphoreType.DMA,
        ],
    )
    def sc_body(ebf_hbm, ef_hbm, idx_hbm, mn_hbm, g_hbm, de_hbm,
                idx_v, rows0, rows1, mn_v, prow_v, de_v, gbuf,
                sem0, sem1, semp0, semp1):
        wid = lax.axis_index("s") * 2 + lax.axis_index("c")
        pltpu.sync_copy(idx_hbm.at[pl.ds(wid * _NODES_W * _RK,
                                         _NODES_W * _RK)], idx_v)
        pltpu.sync_copy(mn_hbm.at[pl.ds(wid * 2 * _PAIRS_W, 2 * _PAIRS_W)],
                        mn_v)
        cpp0 = pltpu.async_copy(ef_hbm.at[mn_v.at[pl.ds(0, _PAIRS_W)]],
                                prow_v.at[pl.ds(0, _PAIRS_W)], semp0)
        cpp1 = pltpu.async_copy(
            ef_hbm.at[mn_v.at[pl.ds(_PAIRS_W, _PAIRS_W)]],
            prow_v.at[pl.ds(_PAIRS_W, _PAIRS_W)], semp1)

        bufs = (rows0, rows1)
        sems = (sem0, sem1)

        def fire(t):
            return pltpu.async_copy(
                ebf_hbm.at[idx_v.at[pl.ds(t * _ROWS, _ROWS)]],
                bufs[t % 2], sems[t % 2])

        cps = {0: fire(0), 1: fire(1)}
        for t in range(_NSTEP):
            cps[t].wait()
            buf = bufs[t % 2]

            def grp_body(g, _, _buf=buf, _t=t):
                u = g // _R
                r = g - u * _R
                base = g * _K

                def row_blk(j, accs, _buf=_buf, _base=base):
                    out = list(accs)
                    for jj in range(4):
                        row = _base + j * 4 + jj
                        for c in range(4):
                            w = _buf[row, pl.ds(c * 16, 16)]
                            v = plsc.bitcast(w, jnp.bfloat16)
                            a, b = plsc.unpack(
                                v, format=plsc.PackFormat.INTERLEAVED)
                            out[2 * c] = out[2 * c] + a
                            out[2 * c + 1] = out[2 * c + 1] + b
                    return tuple(out)

                accs = tuple(jnp.zeros((16,), jnp.float32) for _ in range(8))
                accs = lax.fori_loop(0, _K // 4, row_blk, accs)
                orow = r * _NODES_W + _t * _NPS + u
                for c in range(8):
                    gbuf[orow, pl.ds(c * 16, 16)] = accs[c]
                return 0

            lax.fori_loop(0, _NPS * _R, grp_body, 0)
            if t + 2 < _NSTEP:
                cps[t + 2] = fire(t + 2)

        for r in range(_R):
            pltpu.sync_copy(
                gbuf.at[pl.ds(r * _NODES_W, _NODES_W)],
                g_hbm.at[pl.ds(r * _N + wid * _NODES_W, _NODES_W)])

        cpp0.wait()
        cpp1.wait()

        def pair_body(p, _):
            for c in range(8):
                a = prow_v[2 * p, pl.ds(c * 16, 16)]
                b = prow_v[2 * p + 1, pl.ds(c * 16, 16)]
                de_v[p, pl.ds(c * 16, 16)] = a - b
            return 0

        lax.fori_loop(0, _PAIRS_W, pair_body, 0)
        pltpu.sync_copy(de_v, de_hbm.at[pl.ds(wid * _PAIRS_W, _PAIRS_W)])

    return sc_body


_sc_stage = _make_sc_kernel()


def _dense_stage(ev_ref, g_ref, de_ref, m_ref, n_ref, Wp_ref, Wb_ref, bb_ref,
                 theta_ref, q1_ref, q2_ref, out_ref):
    theta = theta_ref[0, 0]
    alpha = jnp.sum(jnp.exp(-theta * (_CURRENT_TIME - ev_ref[...])))
    x = jnp.zeros((_N, _D), jnp.float32)
    for r in range(_R):
        x = x + jnp.dot(g_ref[r * _N:(r + 1) * _N, :], Wb_ref[r],
                        preferred_element_type=jnp.float32)
    bbar = jnp.mean(bb_ref[...], axis=0, keepdims=True)
    x = x * (1.0 / _RK) + bbar
    s = jnp.mean(jax.nn.sigmoid(x), axis=1, keepdims=True)
    diffs = jnp.dot(de_ref[...], Wp_ref[...],
                    preferred_element_type=jnp.float32)
    d2 = jnp.sum(diffs * diffs, axis=1, keepdims=True)
    q1 = q1_ref[0, 0]
    q2 = q2_ref[0, 0]
    for blk in range(_P // _PAIR_BLK):
        lo, hi = blk * _PAIR_BLK, (blk + 1) * _PAIR_BLK
        iota = lax.broadcasted_iota(jnp.int32, (_PAIR_BLK, _N), 1)
        ohm = (m_ref[lo:hi, :] == iota).astype(jnp.float32)
        ohn = (n_ref[lo:hi, :] == iota).astype(jnp.float32)
        lamn = 0.5 * jnp.dot(ohm + ohn, s, preferred_element_type=jnp.float32)
        lam = -jnp.sqrt(d2[lo:hi, :] + 1e-12) + alpha + lamn
        y = q1 * jnp.exp(lam) + q2 * lam
        out_ref[lo:hi, :] = jax.nn.sigmoid(y)


def kernel(node_pairs, adj_matrix, event_history, neighbor_data, node_embeds,
           W_proj, W_beta, b_beta, decay_theta, q1, q2):
    del adj_matrix
    f32 = jnp.float32
    i32 = jnp.int32
    E = node_embeds.astype(f32)
    T = event_history.shape[0]
    tpad = (-T) % _D
    ev = jnp.concatenate(
        [event_history.astype(f32), jnp.full((tpad,), -1e30, f32)])
    ev = ev.reshape(-1, _D)
    theta = jnp.reshape(decay_theta.astype(f32), (1, 1))
    q1r = jnp.reshape(jnp.asarray(q1, f32), (1, 1))
    q2r = jnp.reshape(jnp.asarray(q2, f32), (1, 1))

    half = jnp.arange(16, dtype=i32)
    intra = jnp.stack([half, half + 16], axis=1).reshape(32)
    perm = (jnp.arange(4, dtype=i32)[:, None] * 32 + intra[None, :]).reshape(
        _D)
    ebf16 = E[:, perm].astype(jnp.bfloat16)
    ebf = jax.lax.bitcast_convert_type(
        ebf16.reshape(_N, _D // 2, 2), jnp.int32)

    idxp = neighbor_data.astype(i32).reshape(-1)
    pairs = node_pairs.astype(i32)
    mnflat = pairs.reshape(-1)
    m = pairs[:, 0:1]
    n = pairs[:, 1:2]

    g, de = _sc_stage(ebf, E, idxp, mnflat)

    out = pl.pallas_call(
        _dense_stage,
        out_shape=jax.ShapeDtypeStruct((_P, 1), f32),
    )(ev, g, de, m, n, W_proj.astype(f32), W_beta.astype(f32),
      b_beta.astype(f32), theta, q1r, q2r)
    return out.reshape(_P)


# speedup vs baseline: 1.3669x; 1.1258x over previous
import functools

import jax
import jax.numpy as jnp
from jax import lax
from jax.experimental import pallas as pl
from jax.experimental.pallas import tpu as pltpu
from jax.experimental.pallas import tpu_sc as plsc

_N = 1024
_D = 128
_P = 4096
_R = 3
_K = 16
_RK = _R * _K
_CURRENT_TIME = 200.0

_NW = 32
_NODES_W = _N // _NW
_PAIRS_W = _P // _NW
_NSTEP = 4
_NPS = _NODES_W // _NSTEP
_ROWS = _NPS * _RK
_PAIR_BLK = 512


def _make_sc_kernel():
    mesh = plsc.VectorSubcoreMesh(core_axis_name="c", subcore_axis_name="s")

    @functools.partial(
        pl.kernel, mesh=mesh,
        compiler_params=pltpu.CompilerParams(
            needs_layout_passes=False, use_tc_tiling_on_sc=False),
        out_type=[jax.ShapeDtypeStruct((_R * _N, _D), jnp.float32),
                  jax.ShapeDtypeStruct((_P, _D), jnp.float32)],
        scratch_types=[
            pltpu.VMEM((_NODES_W * _RK,), jnp.int32),
            pltpu.VMEM((_ROWS, _D // 2), jnp.int32),
            pltpu.VMEM((_ROWS, _D // 2), jnp.int32),
            pltpu.VMEM((2 * _PAIRS_W,), jnp.int32),
            pltpu.VMEM((2 * _PAIRS_W, _D // 2), jnp.int32),
            pltpu.VMEM((_PAIRS_W, _D), jnp.float32),
            pltpu.VMEM((_R * _NODES_W, _D), jnp.float32),
            pltpu.SemaphoreType.DMA,
            pltpu.SemaphoreType.DMA,
            pltpu.SemaphoreType.DMA,
            pltpu.SemaphoreType.DMA,
        ],
    )
    def sc_body(ebf_hbm, idx_hbm, mn_hbm, g_hbm, de_hbm,
                idx_v, rows0, rows1, mn_v, prow_v, de_v, gbuf,
                sem0, sem1, semp0, semp1):
        wid = lax.axis_index("s") * 2 + lax.axis_index("c")
        pltpu.sync_copy(mn_hbm.at[pl.ds(wid * 2 * _PAIRS_W, 2 * _PAIRS_W)],
                        mn_v)
        cpp0 = pltpu.async_copy(ebf_hbm.at[mn_v.at[pl.ds(0, _PAIRS_W)]],
                                prow_v.at[pl.ds(0, _PAIRS_W)], semp0)
        cpp1 = pltpu.async_copy(
            ebf_hbm.at[mn_v.at[pl.ds(_PAIRS_W, _PAIRS_W)]],
            prow_v.at[pl.ds(_PAIRS_W, _PAIRS_W)], semp1)
        pltpu.sync_copy(idx_hbm.at[pl.ds(wid * _NODES_W * _RK,
                                         _NODES_W * _RK)], idx_v)

        bufs = (rows0, rows1)
        sems = (sem0, sem1)

        def fire(t):
            return pltpu.async_copy(
                ebf_hbm.at[idx_v.at[pl.ds(t * _ROWS, _ROWS)]],
                bufs[t % 2], sems[t % 2])

        cps = {0: fire(0), 1: fire(1)}
        for t in range(_NSTEP):
            cps[t].wait()
            buf = bufs[t % 2]

            def grp_body(g, _, _buf=buf, _t=t):
                u = g // _R
                r = g - u * _R
                base = g * _K

                def row_blk(j, accs, _buf=_buf, _base=base):
                    out = list(accs)
                    for jj in range(4):
                        row = _base + j * 4 + jj
                        for c in range(4):
                            w = _buf[row, pl.ds(c * 16, 16)]
                            v = plsc.bitcast(w, jnp.bfloat16)
                            a, b = plsc.unpack(
                                v, format=plsc.PackFormat.INTERLEAVED)
                            out[2 * c] = out[2 * c] + a
                            out[2 * c + 1] = out[2 * c + 1] + b
                    return tuple(out)

                accs = tuple(jnp.zeros((16,), jnp.float32) for _ in range(8))
                accs = lax.fori_loop(0, _K // 4, row_blk, accs)
                orow = r * _NODES_W + _t * _NPS + u
                for c in range(8):
                    gbuf[orow, pl.ds(c * 16, 16)] = accs[c]
                return 0

            lax.fori_loop(0, _NPS * _R, grp_body, 0)
            if t + 2 < _NSTEP:
                cps[t + 2] = fire(t + 2)

        for r in range(_R):
            pltpu.sync_copy(
                gbuf.at[pl.ds(r * _NODES_W, _NODES_W)],
                g_hbm.at[pl.ds(r * _N + wid * _NODES_W, _NODES_W)])

        cpp0.wait()
        cpp1.wait()

        def pair_body(p, _):
            for c in range(4):
                wa = prow_v[2 * p, pl.ds(c * 16, 16)]
                wb = prow_v[2 * p + 1, pl.ds(c * 16, 16)]
                a1, a2 = plsc.unpack(plsc.bitcast(wa, jnp.bfloat16),
                                     format=plsc.PackFormat.INTERLEAVED)
                b1, b2 = plsc.unpack(plsc.bitcast(wb, jnp.bfloat16),
                                     format=plsc.PackFormat.INTERLEAVED)
                de_v[p, pl.ds(c * 32, 16)] = a1 - b1
                de_v[p, pl.ds(c * 32 + 16, 16)] = a2 - b2
            return 0

        lax.fori_loop(0, _PAIRS_W, pair_body, 0)
        pltpu.sync_copy(de_v, de_hbm.at[pl.ds(wid * _PAIRS_W, _PAIRS_W)])

    return sc_body


_sc_stage = _make_sc_kernel()


def _dense_stage(ev_ref, g_ref, de_ref, pr_ref, Wp_ref, Wb_ref, bb_ref,
                 theta_ref, q1_ref, q2_ref, out_ref):
    theta = theta_ref[0, 0]
    alpha = jnp.sum(jnp.exp(-theta * (_CURRENT_TIME - ev_ref[...])))
    x = jnp.zeros((_N, _D), jnp.float32)
    for r in range(_R):
        x = x + jnp.dot(g_ref[r * _N:(r + 1) * _N, :], Wb_ref[r],
                        preferred_element_type=jnp.float32)
    bbar = jnp.mean(bb_ref[...], axis=0, keepdims=True)
    x = x * (1.0 / _RK) + bbar
    s = jnp.mean(jax.nn.sigmoid(x), axis=1, keepdims=True)
    diffs = jnp.dot(de_ref[...], Wp_ref[...],
                    preferred_element_type=jnp.float32)
    d2 = jnp.sum(diffs * diffs, axis=1, keepdims=True)
    q1 = q1_ref[0, 0]
    q2 = q2_ref[0, 0]
    for blk in range(_P // _PAIR_BLK):
        lo, hi = blk * _PAIR_BLK, (blk + 1) * _PAIR_BLK
        iota = lax.broadcasted_iota(jnp.int32, (_PAIR_BLK, _N), 1)
        ohm = (pr_ref[lo:hi, 0:1] == iota).astype(jnp.float32)
        ohn = (pr_ref[lo:hi, 1:2] == iota).astype(jnp.float32)
        lamn = 0.5 * jnp.dot(ohm + ohn, s, preferred_element_type=jnp.float32)
        lam = -jnp.sqrt(d2[lo:hi, :] + 1e-12) + alpha + lamn
        y = q1 * jnp.exp(lam) + q2 * lam
        out_ref[lo:hi, :] = jax.nn.sigmoid(y)


def kernel(node_pairs, adj_matrix, event_history, neighbor_data, node_embeds,
           W_proj, W_beta, b_beta, decay_theta, q1, q2):
    del adj_matrix
    f32 = jnp.float32
    i32 = jnp.int32
    E = node_embeds.astype(f32)
    ev = event_history.astype(f32).reshape(1000, 100)
    theta = jnp.reshape(decay_theta.astype(f32), (1, 1))
    q1r = jnp.reshape(jnp.asarray(q1, f32), (1, 1))
    q2r = jnp.reshape(jnp.asarray(q2, f32), (1, 1))

    half = jnp.arange(16, dtype=i32)
    intra = jnp.stack([half, half + 16], axis=1).reshape(32)
    perm = (jnp.arange(4, dtype=i32)[:, None] * 32 + intra[None, :]).reshape(
        _D)
    ebf16 = E[:, perm].astype(jnp.bfloat16)
    ebf = jax.lax.bitcast_convert_type(
        ebf16.reshape(_N, _D // 2, 2), jnp.int32)

    idxp = neighbor_data.astype(i32).reshape(-1)
    pairs = node_pairs.astype(i32)
    mnflat = pairs.reshape(-1)

    g, de = _sc_stage(ebf, idxp, mnflat)

    out = pl.pallas_call(
        _dense_stage,
        out_shape=jax.ShapeDtypeStruct((_P, 1), f32),
    )(ev, g, de, pairs, W_proj.astype(f32), W_beta.astype(f32),
      b_beta.astype(f32), theta, q1r, q2r)
    return out.reshape(_P)
